# baseline (speedup 1.0000x reference)
import jax
import jax.numpy as jnp
from jax import lax
from jax.experimental import pallas as pl
from jax.experimental.pallas import tpu as pltpu

PAGE = 64
NCHUNK = 8


def _clone_body(scal_ref, seq_smem, pi_smem, seq_ref, pi_ref, tokens_hbm,
                cache_hbm, seq_out, pi_out, tokens_out, cache_out,
                chunk_sems, tok_sem, fix_sem):
    parent = scal_ref[0]
    child = scal_ref[1]
    fresh = scal_ref[2]

    n_pages = cache_hbm.shape[0]
    ch = n_pages // NCHUNK
    bulk = []
    for i in range(NCHUNK):
        c = pltpu.make_async_copy(cache_hbm.at[pl.ds(i * ch, ch)],
                                  cache_out.at[pl.ds(i * ch, ch)],
                                  chunk_sems.at[i])
        c.start()
        bulk.append(c)
    tok_c = pltpu.make_async_copy(tokens_hbm, tokens_out, tok_sem)
    tok_c.start()

    src_len = seq_smem[parent]
    last_idx = jnp.maximum((src_len + PAGE - 1) // PAGE - 1, 0)
    has_partial = jnp.logical_and(src_len % PAGE != 0, src_len > 0)
    src_page = pi_smem[parent, last_idx]

    n_slots = seq_ref.shape[1]
    iota_slot = lax.broadcasted_iota(jnp.int32, (1, n_slots), 1)
    seq_v = seq_ref[...]
    seq_out[...] = jnp.where(iota_slot == child, src_len, seq_v)

    pi_v = pi_ref[...]
    nrow, ncol = pi_v.shape
    row_i = lax.broadcasted_iota(jnp.int32, (nrow, ncol), 0)
    col_i = lax.broadcasted_iota(jnp.int32, (1, ncol), 1)
    parent_row = jnp.sum(jnp.where(row_i == parent, pi_v, 0), axis=0,
                         keepdims=True)
    child_row = jnp.where(jnp.logical_and(col_i == last_idx, has_partial),
                          fresh, parent_row)
    pi_out[...] = jnp.where(row_i == child, child_row, pi_v)

    tok_c.wait()
    row_c = pltpu.make_async_copy(tokens_hbm.at[parent],
                                  tokens_out.at[child], fix_sem)
    row_c.start()
    row_c.wait()

    for c in bulk:
        c.wait()

    @pl.when(has_partial)
    def _():
        pg_c = pltpu.make_async_copy(cache_hbm.at[src_page],
                                     cache_out.at[fresh], fix_sem)
        pg_c.start()
        pg_c.wait()


def kernel(tokens, seq_lens, page_indices, cache, parent_local_id,
           child_local_id, fresh_page):
    scal = jnp.stack([
        jnp.asarray(parent_local_id, jnp.int32),
        jnp.asarray(child_local_id, jnp.int32),
        jnp.asarray(fresh_page, jnp.int32),
    ])
    seq2d = seq_lens.reshape(1, -1)

    out_shapes = (
        jax.ShapeDtypeStruct(seq2d.shape, seq_lens.dtype),
        jax.ShapeDtypeStruct(page_indices.shape, page_indices.dtype),
        jax.ShapeDtypeStruct(tokens.shape, tokens.dtype),
        jax.ShapeDtypeStruct(cache.shape, cache.dtype),
    )
    seq_out, pi_out, tokens_out, cache_out = pl.pallas_call(
        _clone_body,
        out_shape=out_shapes,
        in_specs=[
            pl.BlockSpec(memory_space=pltpu.SMEM),
            pl.BlockSpec(memory_space=pltpu.SMEM),
            pl.BlockSpec(memory_space=pltpu.SMEM),
            pl.BlockSpec(memory_space=pltpu.VMEM),
            pl.BlockSpec(memory_space=pltpu.VMEM),
            pl.BlockSpec(memory_space=pl.ANY),
            pl.BlockSpec(memory_space=pl.ANY),
        ],
        out_specs=[
            pl.BlockSpec(memory_space=pltpu.VMEM),
            pl.BlockSpec(memory_space=pltpu.VMEM),
            pl.BlockSpec(memory_space=pl.ANY),
            pl.BlockSpec(memory_space=pl.ANY),
        ],
        scratch_shapes=[
            pltpu.SemaphoreType.DMA((NCHUNK,)),
            pltpu.SemaphoreType.DMA,
            pltpu.SemaphoreType.DMA,
        ],
    )(scal, seq_lens, page_indices, seq2d, page_indices, tokens, cache)

    return tokens_out, seq_out.reshape(-1), pi_out, cache_out


# speedup vs baseline: 30.2502x; 30.2502x over previous
import jax
import jax.numpy as jnp
from jax import lax
from jax.experimental import pallas as pl
from jax.experimental.pallas import tpu as pltpu

PAGE = 64
BLK = 16


def _src_page(scal, seq, pi):
    parent = scal[0]
    src_len = seq[parent]
    last_idx = jnp.maximum((src_len + PAGE - 1) // PAGE - 1, 0)
    return pi[parent, last_idx]


def _clone_body(scal, seq_sm, pi_sm, cache_blk, srcpg_blk, tokens_in, seq_in,
                pi_in, cache_out, tokens_out, seq_out, pi_out):
    pid = pl.program_id(0)
    parent = scal[0]
    child = scal[1]
    fresh = scal[2]
    src_len = seq_sm[parent]
    last_idx = jnp.maximum((src_len + PAGE - 1) // PAGE - 1, 0)
    has_partial = jnp.logical_and(src_len % PAGE != 0, src_len > 0)

    cache_out[...] = cache_blk[...]

    fresh_here = jnp.logical_and(
        has_partial,
        jnp.logical_and(fresh >= pid * BLK, fresh < (pid + 1) * BLK))

    @pl.when(fresh_here)
    def _():
        cache_out[pl.ds(fresh - pid * BLK, 1)] = srcpg_blk[...]

    @pl.when(pid == 0)
    def _():
        tokens_out[...] = tokens_in[...]
        tokens_out[pl.ds(child, 1), :] = tokens_in[pl.ds(parent, 1), :]

        n_slots = seq_in.shape[1]
        iota_slot = lax.broadcasted_iota(jnp.int32, (1, n_slots), 1)
        seq_out[...] = jnp.where(iota_slot == child, src_len, seq_in[...])

        pi_v = pi_in[...]
        nrow, ncol = pi_v.shape
        row_i = lax.broadcasted_iota(jnp.int32, (nrow, ncol), 0)
        col_i = lax.broadcasted_iota(jnp.int32, (1, ncol), 1)
        parent_row = jnp.sum(jnp.where(row_i == parent, pi_v, 0), axis=0,
                             keepdims=True)
        child_row = jnp.where(
            jnp.logical_and(col_i == last_idx, has_partial), fresh, parent_row)
        pi_out[...] = jnp.where(row_i == child, child_row, pi_v)


def kernel(tokens, seq_lens, page_indices, cache, parent_local_id,
           child_local_id, fresh_page):
    scal = jnp.stack([
        jnp.asarray(parent_local_id, jnp.int32),
        jnp.asarray(child_local_id, jnp.int32),
        jnp.asarray(fresh_page, jnp.int32),
    ])
    seq2d = seq_lens.reshape(1, -1)
    n_pages = cache.shape[0]

    grid_spec = pltpu.PrefetchScalarGridSpec(
        num_scalar_prefetch=3,
        grid=(n_pages // BLK,),
        in_specs=[
            pl.BlockSpec((BLK,) + cache.shape[1:],
                         lambda i, scal, seq, pi: (i, 0, 0)),
            pl.BlockSpec((1,) + cache.shape[1:],
                         lambda i, scal, seq, pi: (_src_page(scal, seq, pi),
                                                   0, 0)),
            pl.BlockSpec(tokens.shape, lambda i, *_: (0, 0)),
            pl.BlockSpec(seq2d.shape, lambda i, *_: (0, 0)),
            pl.BlockSpec(page_indices.shape, lambda i, *_: (0, 0)),
        ],
        out_specs=[
            pl.BlockSpec((BLK,) + cache.shape[1:],
                         lambda i, scal, seq, pi: (i, 0, 0)),
            pl.BlockSpec(tokens.shape, lambda i, *_: (0, 0)),
            pl.BlockSpec(seq2d.shape, lambda i, *_: (0, 0)),
            pl.BlockSpec(page_indices.shape, lambda i, *_: (0, 0)),
        ],
    )
    out_shapes = (
        jax.ShapeDtypeStruct(cache.shape, cache.dtype),
        jax.ShapeDtypeStruct(tokens.shape, tokens.dtype),
        jax.ShapeDtypeStruct(seq2d.shape, seq_lens.dtype),
        jax.ShapeDtypeStruct(page_indices.shape, page_indices.dtype),
    )
    cache_out, tokens_out, seq_out, pi_out = pl.pallas_call(
        _clone_body,
        grid_spec=grid_spec,
        out_shape=out_shapes,
    )(scal, seq_lens, page_indices, cache, cache, tokens, seq2d, page_indices)

    return tokens_out, seq_out.reshape(-1), pi_out, cache_out


# speedup vs baseline: 42.1349x; 1.3929x over previous
import jax
import jax.numpy as jnp
from jax import lax
from jax.experimental import pallas as pl
from jax.experimental.pallas import tpu as pltpu

PAGE = 64
BLK = 32


def _src_page(scal, seq, pi):
    parent = scal[0]
    src_len = seq[parent]
    last_idx = jnp.maximum((src_len + PAGE - 1) // PAGE - 1, 0)
    return pi[parent, last_idx]


def _clone_body(scal, seq_sm, pi_sm, cache_blk, srcpg_blk, tokens_in, seq_in,
                pi_in, cache_out, tokens_out, seq_out, pi_out):
    pid = pl.program_id(0)
    parent = scal[0]
    child = scal[1]
    fresh = scal[2]
    src_len = seq_sm[parent]
    last_idx = jnp.maximum((src_len + PAGE - 1) // PAGE - 1, 0)
    has_partial = jnp.logical_and(src_len % PAGE != 0, src_len > 0)

    cache_out[...] = cache_blk[...]

    fresh_here = jnp.logical_and(
        has_partial,
        jnp.logical_and(fresh >= pid * BLK, fresh < (pid + 1) * BLK))

    @pl.when(fresh_here)
    def _():
        cache_out[pl.ds(fresh - pid * BLK, 1)] = srcpg_blk[...]

    @pl.when(pid == 0)
    def _():
        tokens_out[...] = tokens_in[...]
        tokens_out[pl.ds(child, 1), :] = tokens_in[pl.ds(parent, 1), :]

        n_slots = seq_in.shape[1]
        iota_slot = lax.broadcasted_iota(jnp.int32, (1, n_slots), 1)
        seq_out[...] = jnp.where(iota_slot == child, src_len, seq_in[...])

        pi_v = pi_in[...]
        nrow, ncol = pi_v.shape
        row_i = lax.broadcasted_iota(jnp.int32, (nrow, ncol), 0)
        col_i = lax.broadcasted_iota(jnp.int32, (1, ncol), 1)
        parent_row = jnp.sum(jnp.where(row_i == parent, pi_v, 0), axis=0,
                             keepdims=True)
        child_row = jnp.where(
            jnp.logical_and(col_i == last_idx, has_partial), fresh, parent_row)
        pi_out[...] = jnp.where(row_i == child, child_row, pi_v)


def kernel(tokens, seq_lens, page_indices, cache, parent_local_id,
           child_local_id, fresh_page):
    scal = jnp.stack([
        jnp.asarray(parent_local_id, jnp.int32),
        jnp.asarray(child_local_id, jnp.int32),
        jnp.asarray(fresh_page, jnp.int32),
    ])
    seq2d = seq_lens.reshape(1, -1)
    n_pages = cache.shape[0]

    grid_spec = pltpu.PrefetchScalarGridSpec(
        num_scalar_prefetch=3,
        grid=(n_pages // BLK,),
        in_specs=[
            pl.BlockSpec((BLK,) + cache.shape[1:],
                         lambda i, scal, seq, pi: (i, 0, 0)),
            pl.BlockSpec((1,) + cache.shape[1:],
                         lambda i, scal, seq, pi: (_src_page(scal, seq, pi),
                                                   0, 0)),
            pl.BlockSpec(tokens.shape, lambda i, *_: (0, 0)),
            pl.BlockSpec(seq2d.shape, lambda i, *_: (0, 0)),
            pl.BlockSpec(page_indices.shape, lambda i, *_: (0, 0)),
        ],
        out_specs=[
            pl.BlockSpec((BLK,) + cache.shape[1:],
                         lambda i, scal, seq, pi: (i, 0, 0)),
            pl.BlockSpec(tokens.shape, lambda i, *_: (0, 0)),
            pl.BlockSpec(seq2d.shape, lambda i, *_: (0, 0)),
            pl.BlockSpec(page_indices.shape, lambda i, *_: (0, 0)),
        ],
    )
    out_shapes = (
        jax.ShapeDtypeStruct(cache.shape, cache.dtype),
        jax.ShapeDtypeStruct(tokens.shape, tokens.dtype),
        jax.ShapeDtypeStruct(seq2d.shape, seq_lens.dtype),
        jax.ShapeDtypeStruct(page_indices.shape, page_indices.dtype),
    )
    cache_out, tokens_out, seq_out, pi_out = pl.pallas_call(
        _clone_body,
        grid_spec=grid_spec,
        out_shape=out_shapes,
    )(scal, seq_lens, page_indices, cache, cache, tokens, seq2d, page_indices)

    return tokens_out, seq_out.reshape(-1), pi_out, cache_out


# speedup vs baseline: 45.6004x; 1.0822x over previous
import jax
import jax.numpy as jnp
from jax import lax
from jax.experimental import pallas as pl
from jax.experimental.pallas import tpu as pltpu

PAGE = 64
BPB = 32
NBUF = 8
DEPTH = 4


def _clone_body(scal_ref, seq_sm, pi_sm, seq_in, pi_in, tokens_hbm, cache_hbm,
                seq_out, pi_out, tokens_out, cache_out,
                bufs, tok_buf, srcpg_buf, in_sems, out_sems, tok_sem,
                srcpg_sem):
    parent = scal_ref[0]
    child = scal_ref[1]
    fresh = scal_ref[2]
    src_len = seq_sm[parent]
    last_idx = jnp.maximum((src_len + PAGE - 1) // PAGE - 1, 0)
    has_partial = jnp.logical_and(src_len % PAGE != 0, src_len > 0)
    src_page = pi_sm[parent, last_idx]

    n_pages = cache_hbm.shape[0]
    nblk = n_pages // BPB

    tok_in = pltpu.make_async_copy(tokens_hbm, tok_buf, tok_sem)
    tok_in.start()
    srcpg_in = pltpu.make_async_copy(cache_hbm.at[pl.ds(src_page, 1)],
                                     srcpg_buf, srcpg_sem)
    srcpg_in.start()

    ins = [None] * nblk
    outs = [None] * nblk

    def start_in(i):
        b = i % NBUF
        c = pltpu.make_async_copy(cache_hbm.at[pl.ds(i * BPB, BPB)],
                                  bufs.at[b], in_sems.at[b])
        c.start()
        ins[i] = c

    def drain(j):
        b = j % NBUF
        ins[j].wait()
        blk_has_fresh = jnp.logical_and(
            has_partial,
            jnp.logical_and(fresh >= j * BPB, fresh < (j + 1) * BPB))

        @pl.when(blk_has_fresh)
        def _():
            bufs[b, pl.ds(fresh - j * BPB, 1)] = srcpg_buf[...]

        c = pltpu.make_async_copy(bufs.at[b], cache_out.at[pl.ds(j * BPB, BPB)],
                                  out_sems.at[b])
        c.start()
        outs[j] = c

    srcpg_in.wait()

    for i in range(nblk):
        if i >= NBUF:
            outs[i - NBUF].wait()
        start_in(i)
        if i == 0:
            tok_in.wait()
            row = tok_buf[pl.ds(parent, 1), :]
            tok_buf[pl.ds(child, 1), :] = row
            tok_out = pltpu.make_async_copy(tok_buf, tokens_out, tok_sem)
            tok_out.start()

            n_slots = seq_in.shape[1]
            iota_slot = lax.broadcasted_iota(jnp.int32, (1, n_slots), 1)
            seq_out[...] = jnp.where(iota_slot == child, src_len, seq_in[...])

            pi_v = pi_in[...]
            nrow, ncol = pi_v.shape
            row_i = lax.broadcasted_iota(jnp.int32, (nrow, ncol), 0)
            col_i = lax.broadcasted_iota(jnp.int32, (1, ncol), 1)
            parent_row = jnp.sum(jnp.where(row_i == parent, pi_v, 0), axis=0,
                                 keepdims=True)
            child_row = jnp.where(
                jnp.logical_and(col_i == last_idx, has_partial), fresh,
                parent_row)
            pi_out[...] = jnp.where(row_i == child, child_row, pi_v)
        j = i - DEPTH
        if j >= 0:
            drain(j)
    for j in range(max(nblk - DEPTH, 0), nblk):
        drain(j)
    for j in range(max(nblk - NBUF, 0), nblk):
        outs[j].wait()
    tok_out.wait()


def kernel(tokens, seq_lens, page_indices, cache, parent_local_id,
           child_local_id, fresh_page):
    scal = jnp.stack([
        jnp.asarray(parent_local_id, jnp.int32),
        jnp.asarray(child_local_id, jnp.int32),
        jnp.asarray(fresh_page, jnp.int32),
    ])
    seq2d = seq_lens.reshape(1, -1)

    out_shapes = (
        jax.ShapeDtypeStruct(seq2d.shape, seq_lens.dtype),
        jax.ShapeDtypeStruct(page_indices.shape, page_indices.dtype),
        jax.ShapeDtypeStruct(tokens.shape, tokens.dtype),
        jax.ShapeDtypeStruct(cache.shape, cache.dtype),
    )
    seq_out, pi_out, tokens_out, cache_out = pl.pallas_call(
        _clone_body,
        out_shape=out_shapes,
        in_specs=[
            pl.BlockSpec(memory_space=pltpu.SMEM),
            pl.BlockSpec(memory_space=pltpu.SMEM),
            pl.BlockSpec(memory_space=pltpu.SMEM),
            pl.BlockSpec(memory_space=pltpu.VMEM),
            pl.BlockSpec(memory_space=pltpu.VMEM),
            pl.BlockSpec(memory_space=pl.ANY),
            pl.BlockSpec(memory_space=pl.ANY),
        ],
        out_specs=[
            pl.BlockSpec(memory_space=pltpu.VMEM),
            pl.BlockSpec(memory_space=pltpu.VMEM),
            pl.BlockSpec(memory_space=pl.ANY),
            pl.BlockSpec(memory_space=pl.ANY),
        ],
        scratch_shapes=[
            pltpu.VMEM((NBUF, BPB) + cache.shape[1:], cache.dtype),
            pltpu.VMEM(tokens.shape, tokens.dtype),
            pltpu.VMEM((1,) + cache.shape[1:], cache.dtype),
            pltpu.SemaphoreType.DMA((NBUF,)),
            pltpu.SemaphoreType.DMA((NBUF,)),
            pltpu.SemaphoreType.DMA,
            pltpu.SemaphoreType.DMA,
        ],
    )(scal, seq_lens, page_indices, seq2d, page_indices, tokens, cache)

    return tokens_out, seq_out.reshape(-1), pi_out, cache_out
